# baseline (speedup 1.0000x reference)
import functools
import jax
import jax.numpy as jnp
from jax import lax
from jax.experimental import pallas as pl
from jax.experimental.pallas import tpu as pltpu
from jax.experimental.pallas import tpu_sc as plsc

N = 100000
DH = 16
NNZ = 1600000
LAYERS = 3
NC = 2
NS = 16
CHUNK = 80
BLKE = 2000
NCHK = BLKE // CHUNK
EDGES_PER_TILE = NNZ // NS
NBLK = EDGES_PER_TILE // BLKE
DEPTH = 6
ZCH = 1000
NROWCH = N // ZCH
ZSUB = 200
NSUB = ZCH // ZSUB


def _body(ego0, colv, rowv, valv, out, b1, b2, b3,
          acc, zbuf, cblk, rblk, vblk, grows, stage, gsem, ssem, bsem):
    c = lax.axis_index("c")
    s = lax.axis_index("s")
    ebase = s * EDGES_PER_TILE
    nrow_chunks = jnp.where(s < NROWCH % NS, NROWCH // NS + 1, NROWCH // NS)

    zero = jnp.zeros((DH,), jnp.float32)

    def zinit(i, carry):
        zbuf[i, :] = zero
        return carry

    lax.fori_loop(0, ZSUB, zinit, 0)

    srcs = [ego0, b1, b2]
    dsts = [b1, b2, b3]
    for k in range(LAYERS):
        src = srcs[k]
        dst = dsts[k]

        def zbody(i, carry):
            r0 = (s + i * NS) * ZCH
            for q in range(NSUB):
                pltpu.sync_copy(zbuf, acc.at[pl.ds(r0 + q * ZSUB, ZSUB)])
            return carry

        lax.fori_loop(0, nrow_chunks, zbody, 0)
        plsc.subcore_barrier()

        pltpu.async_copy(colv.at[pl.ds(ebase, BLKE)], cblk.at[0], bsem.at[0])
        pltpu.async_copy(rowv.at[pl.ds(ebase, BLKE)], rblk.at[0], bsem.at[0])
        pltpu.async_copy(valv.at[pl.ds(ebase, BLKE)], vblk.at[0], bsem.at[0])

        def blk_body(b, carry):
            sl = lax.rem(b, 2)
            nsl = 1 - sl
            for _ in range(3):
                pltpu.make_async_copy(colv.at[pl.ds(ebase, BLKE)],
                                      cblk.at[sl], bsem.at[sl]).wait()

            @pl.when(b < NBLK - 1)
            def _stage_next_block():
                nbase = ebase + (b + 1) * BLKE
                pltpu.async_copy(colv.at[pl.ds(nbase, BLKE)],
                                 cblk.at[nsl], bsem.at[nsl])
                pltpu.async_copy(rowv.at[pl.ds(nbase, BLKE)],
                                 rblk.at[nsl], bsem.at[nsl])
                pltpu.async_copy(valv.at[pl.ds(nbase, BLKE)],
                                 vblk.at[nsl], bsem.at[nsl])

            for t in range(DEPTH - 2):
                pltpu.async_copy(
                    src.at[c].at[cblk.at[sl].at[pl.ds(t * CHUNK, CHUNK)]],
                    grows.at[t], gsem.at[t])

            def chunk_body(j, carry2):
                p = lax.rem(j, DEPTH)
                q = lax.rem(j + DEPTH - 2, DEPTH)
                pltpu.make_async_copy(
                    src.at[c].at[cblk.at[sl].at[pl.ds(0, CHUNK)]],
                    grows.at[p], gsem.at[p]).wait()

                def scale(g, carry3):
                    vv = vblk[sl, pl.ds(j * CHUNK + g * DH, DH)]
                    e0 = g * DH
                    for l in range(DH):
                        grows[p, e0 + l, :] = grows[p, e0 + l, :] * vv[l]
                    return carry3

                lax.fori_loop(0, CHUNK // DH, scale, 0)

                @pl.when(j > 1)
                def _drain_scatter_lag2():
                    pltpu.make_async_copy(
                        grows.at[q],
                        acc.at[rblk.at[sl].at[pl.ds(0, CHUNK)]],
                        ssem.at[q]).wait()

                @pl.when(j + DEPTH - 2 < NCHK)
                def _fire_next_gather():
                    pltpu.async_copy(
                        src.at[c].at[cblk.at[sl].at[
                            pl.ds((j + DEPTH - 2) * CHUNK, CHUNK)]],
                        grows.at[q], gsem.at[q])

                pltpu.async_copy(grows.at[p],
                                 acc.at[rblk.at[sl].at[pl.ds(j * CHUNK,
                                                             CHUNK)]],
                                 ssem.at[p], add=True)
                return carry2

            lax.fori_loop(0, NCHK, chunk_body, 0)
            for t in (NCHK - 2, NCHK - 1):
                pltpu.make_async_copy(
                    grows.at[t % DEPTH],
                    acc.at[rblk.at[sl].at[pl.ds(0, CHUNK)]],
                    ssem.at[t % DEPTH]).wait()
            return carry

        lax.fori_loop(0, NBLK, blk_body, 0)
        plsc.subcore_barrier()

        def wbody(i, carry):
            r0 = (s + i * NS) * ZCH
            pltpu.sync_copy(acc.at[pl.ds(r0, ZCH)],
                            dst.at[c].at[pl.ds(r0, ZCH)])
            return carry

        lax.fori_loop(0, nrow_chunks, wbody, 0)
        plsc.subcore_barrier()

    def mbody(i, carry):
        r0 = (s + i * NS) * ZCH
        for q in range(NSUB):
            rq = r0 + q * ZSUB
            pltpu.sync_copy(ego0.at[c].at[pl.ds(rq, ZSUB)], stage.at[0])
            for lay in (b1, b2, b3):
                pltpu.sync_copy(lay.at[c].at[pl.ds(rq, ZSUB)], stage.at[1])

                def add_row(r, carry2):
                    stage[0, r, :] = stage[0, r, :] + stage[1, r, :]
                    return carry2

                lax.fori_loop(0, ZSUB, add_row, 0)

            def scale_row(r, carry2):
                stage[0, r, :] = stage[0, r, :] * 0.25
                return carry2

            lax.fori_loop(0, ZSUB, scale_row, 0)
            pltpu.sync_copy(stage.at[0], out.at[c].at[pl.ds(rq, ZSUB)])
        return carry

    lax.fori_loop(0, nrow_chunks, mbody, 0)


_sc_call = functools.partial(
    pl.kernel,
    mesh=plsc.VectorSubcoreMesh(core_axis_name="c", subcore_axis_name="s"),
    compiler_params=pltpu.CompilerParams(use_tc_tiling_on_sc=False),
    out_type=[jax.ShapeDtypeStruct((NC, N, DH), jnp.float32)] * 4,
    scratch_types=[
        pltpu.VMEM_SHARED((N, DH), jnp.float32),
        pltpu.VMEM((ZSUB, DH), jnp.float32),
        pltpu.VMEM((2, BLKE), jnp.int32),
        pltpu.VMEM((2, BLKE), jnp.int32),
        pltpu.VMEM((2, BLKE), jnp.float32),
        pltpu.VMEM((DEPTH, CHUNK, DH), jnp.float32),
        pltpu.VMEM((2, ZSUB, DH), jnp.float32),
        pltpu.SemaphoreType.DMA((DEPTH,)),
        pltpu.SemaphoreType.DMA((DEPTH,)),
        pltpu.SemaphoreType.DMA((2,)),
    ],
)(_body)


@jax.jit
def kernel(user_emb, item_emb, adj_row, adj_col, adj_val):
    ego = jnp.concatenate([user_emb, item_emb], axis=0)
    ego0 = jnp.stack([ego[:, :DH], ego[:, DH:]], axis=0)
    out, _, _, _ = _sc_call(ego0, adj_col, adj_row, adj_val)
    return jnp.concatenate([out[0], out[1]], axis=1)


# speedup vs baseline: 1.3892x; 1.3892x over previous
import functools
import jax
import jax.numpy as jnp
from jax import lax
from jax.experimental import pallas as pl
from jax.experimental.pallas import tpu as pltpu
from jax.experimental.pallas import tpu_sc as plsc

N = 100000
DH = 16
NNZ = 1600000
LAYERS = 3
NC = 2
NS = 16
CHUNK = 80
BLKE = 2000
NCHK = BLKE // CHUNK
EDGES_PER_TILE = NNZ // NS
NBLK = EDGES_PER_TILE // BLKE
DEPTH = 4
ZCH = 1000
NROWCH = N // ZCH
ZSUB = 200
NSUB = ZCH // ZSUB


def _body(ego0, colv, rowv, valv, out, b1, b2, b3,
          acc, zbuf, cblk, rblk, vblk, grows, stage, gsem, ssem, bsem):
    c = lax.axis_index("c")
    s = lax.axis_index("s")
    ebase = s * EDGES_PER_TILE
    nrow_chunks = jnp.where(s < NROWCH % NS, NROWCH // NS + 1, NROWCH // NS)

    zero = jnp.zeros((DH,), jnp.float32)

    def zinit(i, carry):
        zbuf[i, :] = zero
        return carry

    lax.fori_loop(0, ZSUB, zinit, 0)

    srcs = [ego0, b1, b2]
    dsts = [b1, b2, b3]
    for k in range(LAYERS):
        src = srcs[k]
        dst = dsts[k]

        if k == 0:
            def zbody(i, carry):
                r0 = (s + i * NS) * ZCH
                for q in range(NSUB):
                    pltpu.sync_copy(zbuf, acc.at[pl.ds(r0 + q * ZSUB, ZSUB)])
                return carry

            lax.fori_loop(0, nrow_chunks, zbody, 0)
        plsc.subcore_barrier()

        pltpu.async_copy(colv.at[pl.ds(ebase, BLKE)], cblk.at[0], bsem.at[0])
        pltpu.async_copy(rowv.at[pl.ds(ebase, BLKE)], rblk.at[0], bsem.at[0])
        pltpu.async_copy(valv.at[pl.ds(ebase, BLKE)], vblk.at[0], bsem.at[0])

        def blk_body(b, carry):
            sl = lax.rem(b, 2)
            nsl = 1 - sl
            for _ in range(3):
                pltpu.make_async_copy(colv.at[pl.ds(ebase, BLKE)],
                                      cblk.at[sl], bsem.at[sl]).wait()

            @pl.when(b < NBLK - 1)
            def _stage_next_block():
                nbase = ebase + (b + 1) * BLKE
                pltpu.async_copy(colv.at[pl.ds(nbase, BLKE)],
                                 cblk.at[nsl], bsem.at[nsl])
                pltpu.async_copy(rowv.at[pl.ds(nbase, BLKE)],
                                 rblk.at[nsl], bsem.at[nsl])
                pltpu.async_copy(valv.at[pl.ds(nbase, BLKE)],
                                 vblk.at[nsl], bsem.at[nsl])

            for t in range(DEPTH - 1):
                pltpu.async_copy(
                    src.at[c].at[cblk.at[sl].at[pl.ds(t * CHUNK, CHUNK)]],
                    grows.at[t], gsem.at[t])

            def chunk_body(j, carry2):
                p = lax.rem(j, DEPTH)
                q = lax.rem(j + DEPTH - 1, DEPTH)
                pltpu.make_async_copy(
                    src.at[c].at[cblk.at[sl].at[pl.ds(0, CHUNK)]],
                    grows.at[p], gsem.at[p]).wait()

                def scale(g, carry3):
                    vv = vblk[sl, pl.ds(j * CHUNK + g * DH, DH)]
                    e0 = g * DH
                    for l in range(DH):
                        grows[p, e0 + l, :] = grows[p, e0 + l, :] * vv[l]
                    return carry3

                lax.fori_loop(0, CHUNK // DH, scale, 0)

                @pl.when(j > 0)
                def _drain_prev_scatter():
                    pltpu.make_async_copy(
                        grows.at[q],
                        acc.at[rblk.at[sl].at[pl.ds(0, CHUNK)]], ssem).wait()

                @pl.when(j + DEPTH - 1 < NCHK)
                def _fire_next_gather():
                    pltpu.async_copy(
                        src.at[c].at[cblk.at[sl].at[
                            pl.ds((j + DEPTH - 1) * CHUNK, CHUNK)]],
                        grows.at[q], gsem.at[q])

                pltpu.async_copy(grows.at[p],
                                 acc.at[rblk.at[sl].at[pl.ds(j * CHUNK,
                                                             CHUNK)]],
                                 ssem, add=True)
                return carry2

            lax.fori_loop(0, NCHK, chunk_body, 0)
            pltpu.make_async_copy(
                grows.at[(NCHK - 1) % DEPTH],
                acc.at[rblk.at[sl].at[pl.ds(0, CHUNK)]], ssem).wait()
            return carry

        lax.fori_loop(0, NBLK, blk_body, 0)
        plsc.subcore_barrier()

        def wbody(i, carry):
            r0 = (s + i * NS) * ZCH
            pltpu.sync_copy(acc.at[pl.ds(r0, ZCH)],
                            dst.at[c].at[pl.ds(r0, ZCH)])
            if k < LAYERS - 1:
                for q in range(NSUB):
                    pltpu.sync_copy(zbuf, acc.at[pl.ds(r0 + q * ZSUB, ZSUB)])
            return carry

        lax.fori_loop(0, nrow_chunks, wbody, 0)
        plsc.subcore_barrier()

    def mbody(i, carry):
        r0 = (s + i * NS) * ZCH
        for q in range(NSUB):
            rq = r0 + q * ZSUB
            pltpu.sync_copy(ego0.at[c].at[pl.ds(rq, ZSUB)], stage.at[0])
            for lay in (b1, b2, b3):
                pltpu.sync_copy(lay.at[c].at[pl.ds(rq, ZSUB)], stage.at[1])

                def add_row(r, carry2):
                    stage[0, r, :] = stage[0, r, :] + stage[1, r, :]
                    return carry2

                lax.fori_loop(0, ZSUB, add_row, 0)

            def scale_row(r, carry2):
                stage[0, r, :] = stage[0, r, :] * 0.25
                return carry2

            lax.fori_loop(0, ZSUB, scale_row, 0)
            pltpu.sync_copy(stage.at[0], out.at[c].at[pl.ds(rq, ZSUB)])
        return carry

    lax.fori_loop(0, nrow_chunks, mbody, 0)


_sc_call = functools.partial(
    pl.kernel,
    mesh=plsc.VectorSubcoreMesh(core_axis_name="c", subcore_axis_name="s"),
    compiler_params=pltpu.CompilerParams(use_tc_tiling_on_sc=False),
    out_type=[jax.ShapeDtypeStruct((NC, N, DH), jnp.float32)] * 4,
    scratch_types=[
        pltpu.VMEM_SHARED((N, DH), jnp.float32),
        pltpu.VMEM((ZSUB, DH), jnp.float32),
        pltpu.VMEM((2, BLKE), jnp.int32),
        pltpu.VMEM((2, BLKE), jnp.int32),
        pltpu.VMEM((2, BLKE), jnp.float32),
        pltpu.VMEM((DEPTH, CHUNK, DH), jnp.float32),
        pltpu.VMEM((2, ZSUB, DH), jnp.float32),
        pltpu.SemaphoreType.DMA((DEPTH,)),
        pltpu.SemaphoreType.DMA,
        pltpu.SemaphoreType.DMA((2,)),
    ],
)(_body)


@jax.jit
def kernel(user_emb, item_emb, adj_row, adj_col, adj_val):
    ego = jnp.concatenate([user_emb, item_emb], axis=0)
    ego0 = jnp.stack([ego[:, :DH], ego[:, DH:]], axis=0)
    out, _, _, _ = _sc_call(ego0, adj_col, adj_row, adj_val)
    return jnp.concatenate([out[0], out[1]], axis=1)
